# baseline (speedup 1.0000x reference)
import functools

import jax
import jax.numpy as jnp
from jax import lax
from jax.experimental import pallas as pl
from jax.experimental.pallas import tpu as pltpu
from jax.experimental.pallas import tpu_sc as plsc

DIM = 256
N_EMBED = 8192
COMMITMENT_COST = 0.25
TM = 2048
TCK = 1024
KAUG = DIM + 8
SC_CORES = 2
SC_SUBCORES = 16
SC_WORKERS = SC_CORES * SC_SUBCORES


def _prep_w_body(w_ref, waug_ref):
    w = w_ref[...]
    w2 = jnp.sum(w * w, axis=0, keepdims=True)
    hi = w2.astype(jnp.bfloat16)
    lo = (w2 - hi.astype(jnp.float32)).astype(jnp.bfloat16)
    waug_ref[0:DIM, :] = w.astype(jnp.bfloat16)
    waug_ref[DIM:DIM + 1, :] = hi
    waug_ref[DIM + 1:DIM + 2, :] = lo
    waug_ref[DIM + 2:KAUG, :] = jnp.zeros((6, N_EMBED), jnp.bfloat16)


def _prep_x_body(x_ref, xaug_ref, x2_ref):
    i = pl.program_id(0)

    @pl.when(i == 0)
    def _init():
        x2_ref[...] = jnp.zeros_like(x2_ref)

    x = x_ref[...]
    xaug_ref[:, 0:DIM] = (-2.0 * x).astype(jnp.bfloat16)
    xaug_ref[:, DIM:DIM + 2] = jnp.ones((x.shape[0], 2), jnp.bfloat16)
    xaug_ref[:, DIM + 2:KAUG] = jnp.zeros((x.shape[0], 6), jnp.bfloat16)
    x2_ref[...] += jnp.reshape(jnp.sum(x * x), (1, 1))


def _vq_body(xa_ref, waug_ref, idx_ref, acc_ref):
    i = pl.program_id(0)
    ni = pl.num_programs(0)

    @pl.when(i == 0)
    def _init():
        acc_ref[...] = jnp.zeros_like(acc_ref)

    xa = xa_ref[...]
    m = jnp.full((TM, 1), jnp.inf, jnp.float32)
    idx = jnp.zeros((TM, 1), jnp.float32)
    io = lax.broadcasted_iota(jnp.int32, (TM, TCK), 1).astype(jnp.float32)
    for c0 in range(0, N_EMBED, TCK):
        wc = waug_ref[:, c0:c0 + TCK]
        s = jnp.dot(xa, wc, preferred_element_type=jnp.float32)
        mc = jnp.min(s, axis=1, keepdims=True)
        ic = jnp.min(jnp.where(s == mc, io, jnp.float32(N_EMBED)),
                     axis=1, keepdims=True) + jnp.float32(c0)
        upd = mc < m
        idx = jnp.where(upd, ic, idx)
        m = jnp.where(upd, mc, m)
    idxt = jnp.swapaxes(idx.astype(jnp.int32), 0, 1)
    idx_ref[...] = idxt[None]
    acc_ref[...] += jnp.reshape(jnp.sum(m), (1, 1))


def _fin_body(counts_ref, acc_ref, x2_ref, loss_ref, perp_ref):
    total = jnp.float32(32768.0)
    p = jnp.sum(counts_ref[...], axis=0, keepdims=True) / total
    ent = jnp.sum(p * jnp.log(p + 1e-10))
    perp_ref[...] = jnp.reshape(jnp.exp(-ent), (1, 1))
    loss_ref[...] = (COMMITMENT_COST
                     * (acc_ref[...] + x2_ref[...]) / (total * DIM))


def _sc_hist(idx_hbm, out_hbm, idx_v, ones_v, z_v, hist_sh, sem):
    cid = lax.axis_index("c")
    sid = lax.axis_index("s")
    wid = cid * SC_SUBCORES + sid
    rows_per_w = idx_hbm.shape[0] // SC_WORKERS
    hist_slice = N_EMBED // SC_SUBCORES

    @pl.loop(0, hist_slice, step=16)
    def _z(j):
        z_v[pl.ds(j, 16)] = jnp.zeros((16,), jnp.float32)

    pltpu.sync_copy(z_v, hist_sh.at[pl.ds(sid * hist_slice, hist_slice)])

    @pl.loop(0, 128, step=16)
    def _o(j):
        ones_v[pl.ds(j, 16)] = jnp.ones((16,), jnp.float32)

    pltpu.async_copy(idx_hbm.at[pl.ds(wid * rows_per_w, rows_per_w)],
                     idx_v, sem).wait()
    plsc.subcore_barrier()

    for j in range(8):
        pltpu.sync_copy(ones_v, hist_sh.at[idx_v.at[j]], add=True)

    plsc.subcore_barrier()
    pltpu.sync_copy(hist_sh.at[pl.ds(sid * hist_slice, hist_slice)],
                    out_hbm.at[cid, pl.ds(sid * hist_slice, hist_slice)])


def kernel(input, w):
    x = input.reshape(-1, DIM)
    n = x.shape[0]
    ni = n // TM

    waug = pl.pallas_call(
        _prep_w_body,
        in_specs=[pl.BlockSpec((DIM, N_EMBED), lambda: (0, 0))],
        out_specs=pl.BlockSpec((KAUG, N_EMBED), lambda: (0, 0)),
        out_shape=jax.ShapeDtypeStruct((KAUG, N_EMBED), jnp.bfloat16),
    )(w)

    xaug, x2sum = pl.pallas_call(
        _prep_x_body,
        grid=(ni,),
        in_specs=[pl.BlockSpec((TM, DIM), lambda i: (i, 0))],
        out_specs=[
            pl.BlockSpec((TM, KAUG), lambda i: (i, 0)),
            pl.BlockSpec((1, 1), lambda i: (0, 0)),
        ],
        out_shape=[
            jax.ShapeDtypeStruct((n, KAUG), jnp.bfloat16),
            jax.ShapeDtypeStruct((1, 1), jnp.float32),
        ],
    )(x)

    idx3, acc = pl.pallas_call(
        _vq_body,
        grid=(ni,),
        in_specs=[
            pl.BlockSpec((TM, KAUG), lambda i: (i, 0)),
            pl.BlockSpec((KAUG, N_EMBED), lambda i: (0, 0)),
        ],
        out_specs=[
            pl.BlockSpec((1, 1, TM), lambda i: (i, 0, 0)),
            pl.BlockSpec((1, 1), lambda i: (0, 0)),
        ],
        out_shape=[
            jax.ShapeDtypeStruct((ni, 1, TM), jnp.int32),
            jax.ShapeDtypeStruct((1, 1), jnp.float32),
        ],
    )(xaug, waug)

    idx2 = idx3.reshape(n // 128, 128)

    hist_kernel = functools.partial(
        pl.kernel,
        mesh=plsc.VectorSubcoreMesh(core_axis_name="c", subcore_axis_name="s"),
        out_type=jax.ShapeDtypeStruct((SC_CORES, N_EMBED), jnp.float32),
        scratch_types=[
            pltpu.VMEM((n // SC_WORKERS // 128, 128), jnp.int32),
            pltpu.VMEM((128,), jnp.float32),
            pltpu.VMEM((N_EMBED // SC_SUBCORES,), jnp.float32),
            pltpu.VMEM_SHARED((N_EMBED,), jnp.float32),
            pltpu.SemaphoreType.DMA,
        ],
    )(_sc_hist)
    counts = hist_kernel(idx2)

    loss, perp = pl.pallas_call(
        _fin_body,
        in_specs=[
            pl.BlockSpec((SC_CORES, N_EMBED), lambda: (0, 0)),
            pl.BlockSpec((1, 1), lambda: (0, 0)),
            pl.BlockSpec((1, 1), lambda: (0, 0)),
        ],
        out_specs=[
            pl.BlockSpec((1, 1), lambda: (0, 0)),
            pl.BlockSpec((1, 1), lambda: (0, 0)),
        ],
        out_shape=[
            jax.ShapeDtypeStruct((1, 1), jnp.float32),
            jax.ShapeDtypeStruct((1, 1), jnp.float32),
        ],
    )(counts, acc, x2sum)
    return loss[0, 0], perp[0, 0]


# speedup vs baseline: 1.1063x; 1.1063x over previous
import functools

import jax
import jax.numpy as jnp
from jax import lax
from jax.experimental import pallas as pl
from jax.experimental.pallas import tpu as pltpu
from jax.experimental.pallas import tpu_sc as plsc

DIM = 256
N_EMBED = 8192
COMMITMENT_COST = 0.25
TM = 2048
TCK = 256
KAUG = DIM + 8
SC_CORES = 2
SC_SUBCORES = 16
SC_WORKERS = SC_CORES * SC_SUBCORES


def _prep_w_body(w_ref, waug_ref):
    w = w_ref[...]
    w2 = jnp.sum(w * w, axis=0, keepdims=True)
    hi = w2.astype(jnp.bfloat16)
    lo = (w2 - hi.astype(jnp.float32)).astype(jnp.bfloat16)
    waug_ref[0:DIM, :] = w.astype(jnp.bfloat16)
    waug_ref[DIM:DIM + 1, :] = hi
    waug_ref[DIM + 1:DIM + 2, :] = lo
    waug_ref[DIM + 2:KAUG, :] = jnp.zeros((6, N_EMBED), jnp.bfloat16)


def _prep_x_body(x_ref, xaug_ref, x2_ref):
    i = pl.program_id(0)

    @pl.when(i == 0)
    def _init():
        x2_ref[...] = jnp.zeros_like(x2_ref)

    x = x_ref[...]
    xaug_ref[:, 0:DIM] = (-2.0 * x).astype(jnp.bfloat16)
    xaug_ref[:, DIM:DIM + 2] = jnp.ones((x.shape[0], 2), jnp.bfloat16)
    xaug_ref[:, DIM + 2:KAUG] = jnp.zeros((x.shape[0], 6), jnp.bfloat16)
    x2_ref[...] += jnp.reshape(jnp.sum(x * x), (1, 1))


def _vq_body(xa_ref, waug_ref, idx_ref, acc_ref):
    i = pl.program_id(0)
    ni = pl.num_programs(0)

    @pl.when(i == 0)
    def _init():
        acc_ref[...] = jnp.zeros_like(acc_ref)

    xa = xa_ref[...]
    m = jnp.full((TM, 1), jnp.inf, jnp.float32)
    idx = jnp.zeros((TM, 1), jnp.float32)
    io = lax.broadcasted_iota(jnp.int32, (TM, TCK), 1).astype(jnp.float32)
    for c0 in range(0, N_EMBED, TCK):
        wc = waug_ref[:, c0:c0 + TCK]
        s = jnp.dot(xa, wc, preferred_element_type=jnp.float32)
        mc = jnp.min(s, axis=1, keepdims=True)
        ic = jnp.min(jnp.where(s == mc, io, jnp.float32(N_EMBED)),
                     axis=1, keepdims=True) + jnp.float32(c0)
        upd = mc < m
        idx = jnp.where(upd, ic, idx)
        m = jnp.where(upd, mc, m)
    idxt = jnp.swapaxes(idx.astype(jnp.int32), 0, 1)
    idx_ref[...] = idxt[None]
    acc_ref[...] += jnp.reshape(jnp.sum(m), (1, 1))


def _fin_body(counts_ref, acc_ref, x2_ref, loss_ref, perp_ref):
    total = jnp.float32(32768.0)
    p = jnp.sum(counts_ref[...], axis=0, keepdims=True) / total
    ent = jnp.sum(p * jnp.log(p + 1e-10))
    perp_ref[...] = jnp.reshape(jnp.exp(-ent), (1, 1))
    loss_ref[...] = (COMMITMENT_COST
                     * (acc_ref[...] + x2_ref[...]) / (total * DIM))


def _sc_hist(idx_hbm, out_hbm, idx_v, ones_v, z_v, hist_sh, sem):
    cid = lax.axis_index("c")
    sid = lax.axis_index("s")
    wid = cid * SC_SUBCORES + sid
    rows_per_w = idx_hbm.shape[0] // SC_WORKERS
    hist_slice = N_EMBED // SC_SUBCORES

    @pl.loop(0, hist_slice, step=16)
    def _z(j):
        z_v[pl.ds(j, 16)] = jnp.zeros((16,), jnp.float32)

    pltpu.sync_copy(z_v, hist_sh.at[pl.ds(sid * hist_slice, hist_slice)])

    @pl.loop(0, 128, step=16)
    def _o(j):
        ones_v[pl.ds(j, 16)] = jnp.ones((16,), jnp.float32)

    pltpu.async_copy(idx_hbm.at[pl.ds(wid * rows_per_w, rows_per_w)],
                     idx_v, sem).wait()
    plsc.subcore_barrier()

    for j in range(8):
        pltpu.sync_copy(ones_v, hist_sh.at[idx_v.at[j]], add=True)

    plsc.subcore_barrier()
    pltpu.sync_copy(hist_sh.at[pl.ds(sid * hist_slice, hist_slice)],
                    out_hbm.at[cid, pl.ds(sid * hist_slice, hist_slice)])


def kernel(input, w):
    x = input.reshape(-1, DIM)
    n = x.shape[0]
    ni = n // TM

    waug = pl.pallas_call(
        _prep_w_body,
        in_specs=[pl.BlockSpec((DIM, N_EMBED), lambda: (0, 0))],
        out_specs=pl.BlockSpec((KAUG, N_EMBED), lambda: (0, 0)),
        out_shape=jax.ShapeDtypeStruct((KAUG, N_EMBED), jnp.bfloat16),
    )(w)

    xaug, x2sum = pl.pallas_call(
        _prep_x_body,
        grid=(ni,),
        in_specs=[pl.BlockSpec((TM, DIM), lambda i: (i, 0))],
        out_specs=[
            pl.BlockSpec((TM, KAUG), lambda i: (i, 0)),
            pl.BlockSpec((1, 1), lambda i: (0, 0)),
        ],
        out_shape=[
            jax.ShapeDtypeStruct((n, KAUG), jnp.bfloat16),
            jax.ShapeDtypeStruct((1, 1), jnp.float32),
        ],
    )(x)

    idx3, acc = pl.pallas_call(
        _vq_body,
        grid=(ni,),
        in_specs=[
            pl.BlockSpec((TM, KAUG), lambda i: (i, 0)),
            pl.BlockSpec((KAUG, N_EMBED), lambda i: (0, 0)),
        ],
        out_specs=[
            pl.BlockSpec((1, 1, TM), lambda i: (i, 0, 0)),
            pl.BlockSpec((1, 1), lambda i: (0, 0)),
        ],
        out_shape=[
            jax.ShapeDtypeStruct((ni, 1, TM), jnp.int32),
            jax.ShapeDtypeStruct((1, 1), jnp.float32),
        ],
    )(xaug, waug)

    idx2 = idx3.reshape(n // 128, 128)

    hist_kernel = functools.partial(
        pl.kernel,
        mesh=plsc.VectorSubcoreMesh(core_axis_name="c", subcore_axis_name="s"),
        out_type=jax.ShapeDtypeStruct((SC_CORES, N_EMBED), jnp.float32),
        scratch_types=[
            pltpu.VMEM((n // SC_WORKERS // 128, 128), jnp.int32),
            pltpu.VMEM((128,), jnp.float32),
            pltpu.VMEM((N_EMBED // SC_SUBCORES,), jnp.float32),
            pltpu.VMEM_SHARED((N_EMBED,), jnp.float32),
            pltpu.SemaphoreType.DMA,
        ],
    )(_sc_hist)
    counts = hist_kernel(idx2)

    loss, perp = pl.pallas_call(
        _fin_body,
        in_specs=[
            pl.BlockSpec((SC_CORES, N_EMBED), lambda: (0, 0)),
            pl.BlockSpec((1, 1), lambda: (0, 0)),
            pl.BlockSpec((1, 1), lambda: (0, 0)),
        ],
        out_specs=[
            pl.BlockSpec((1, 1), lambda: (0, 0)),
            pl.BlockSpec((1, 1), lambda: (0, 0)),
        ],
        out_shape=[
            jax.ShapeDtypeStruct((1, 1), jnp.float32),
            jax.ShapeDtypeStruct((1, 1), jnp.float32),
        ],
    )(counts, acc, x2sum)
    return loss[0, 0], perp[0, 0]
